# baseline (speedup 1.0000x reference)
import jax
import jax.numpy as jnp
from jax.experimental import pallas as pl
from jax.experimental.pallas import tpu as pltpu

BM = 2048
BN = 2048
BK = 256


def _matmul_kernel(x_ref, w_ref, b_ref, o_ref):
    acc = jnp.dot(x_ref[...], w_ref[...], preferred_element_type=jnp.float32)

    @pl.when(pl.program_id(2) == 0)
    def _first():
        o_ref[...] = acc + b_ref[...]

    @pl.when(pl.program_id(2) != 0)
    def _rest():
        o_ref[...] = o_ref[...] + acc


def kernel(inputs, W, b):
    M, K = inputs.shape
    _, N = W.shape
    b2d = b.reshape(1, N)

    grid = (M // BM, N // BN, K // BK)
    out = pl.pallas_call(
        _matmul_kernel,
        grid=grid,
        in_specs=[
            pl.BlockSpec((BM, BK), lambda i, j, k: (i, k)),
            pl.BlockSpec((BK, BN), lambda i, j, k: (k, j)),
            pl.BlockSpec((1, BN), lambda i, j, k: (0, j)),
        ],
        out_specs=pl.BlockSpec((BM, BN), lambda i, j, k: (i, j)),
        out_shape=jax.ShapeDtypeStruct((M, N), jnp.float32),
        compiler_params=pltpu.CompilerParams(
            dimension_semantics=("parallel", "parallel", "arbitrary"),
        ),
    )(inputs, W, b2d)
    return out


# speedup vs baseline: 1.4109x; 1.4109x over previous
import jax
import jax.numpy as jnp
from jax.experimental import pallas as pl
from jax.experimental.pallas import tpu as pltpu

BM = 2048
BN = 1024
BK = 1024


def _matmul_kernel(x_ref, w_ref, b_ref, o_ref):
    acc = jnp.dot(
        x_ref[...].astype(jnp.bfloat16),
        w_ref[...].astype(jnp.bfloat16),
        preferred_element_type=jnp.float32,
    )

    @pl.when(pl.program_id(2) == 0)
    def _first():
        o_ref[...] = acc + b_ref[...]

    @pl.when(pl.program_id(2) != 0)
    def _rest():
        o_ref[...] = o_ref[...] + acc


def kernel(inputs, W, b):
    M, K = inputs.shape
    _, N = W.shape
    b2d = b.reshape(1, N)

    grid = (M // BM, N // BN, K // BK)
    out = pl.pallas_call(
        _matmul_kernel,
        grid=grid,
        in_specs=[
            pl.BlockSpec((BM, BK), lambda i, j, k: (i, k)),
            pl.BlockSpec((BK, BN), lambda i, j, k: (k, j)),
            pl.BlockSpec((1, BN), lambda i, j, k: (0, j)),
        ],
        out_specs=pl.BlockSpec((BM, BN), lambda i, j, k: (i, j)),
        out_shape=jax.ShapeDtypeStruct((M, N), jnp.float32),
        compiler_params=pltpu.CompilerParams(
            dimension_semantics=("parallel", "parallel", "arbitrary"),
        ),
    )(inputs, W, b2d)
    return out


# speedup vs baseline: 1.4156x; 1.0034x over previous
import jax
import jax.numpy as jnp
from jax.experimental import pallas as pl
from jax.experimental.pallas import tpu as pltpu

BM = 1024
BN = 1024


def _matmul_kernel(x_ref, w_ref, b_ref, o_ref):
    o_ref[...] = (
        jnp.dot(x_ref[...], w_ref[...], preferred_element_type=jnp.float32)
        + b_ref[...]
    )


def kernel(inputs, W, b):
    M, K = inputs.shape
    _, N = W.shape
    b2d = b.reshape(1, N)
    x16 = inputs.astype(jnp.bfloat16)
    w16 = W.astype(jnp.bfloat16)

    grid = (M // BM, N // BN)
    out = pl.pallas_call(
        _matmul_kernel,
        grid=grid,
        in_specs=[
            pl.BlockSpec((BM, K), lambda i, j: (i, 0)),
            pl.BlockSpec((K, BN), lambda i, j: (0, j)),
            pl.BlockSpec((1, BN), lambda i, j: (0, j)),
        ],
        out_specs=pl.BlockSpec((BM, BN), lambda i, j: (i, j)),
        out_shape=jax.ShapeDtypeStruct((M, N), jnp.float32),
        compiler_params=pltpu.CompilerParams(
            dimension_semantics=("parallel", "parallel"),
        ),
    )(x16, w16, b2d)
    return out


# speedup vs baseline: 1.5129x; 1.0687x over previous
import jax
import jax.numpy as jnp
from jax.experimental import pallas as pl
from jax.experimental.pallas import tpu as pltpu

BM = 1024
BN = 512


def _matmul_kernel(x_ref, w_ref, b_ref, o_ref):
    o_ref[...] = (
        jnp.dot(
            x_ref[...].astype(jnp.bfloat16),
            w_ref[...],
            preferred_element_type=jnp.float32,
        )
        + b_ref[...]
    )


def kernel(inputs, W, b):
    M, K = inputs.shape
    _, N = W.shape
    b2d = b.reshape(1, N)
    w16 = W.astype(jnp.bfloat16)

    grid = (M // BM, N // BN)
    out = pl.pallas_call(
        _matmul_kernel,
        grid=grid,
        in_specs=[
            pl.BlockSpec((BM, K), lambda i, j: (i, 0)),
            pl.BlockSpec((K, BN), lambda i, j: (0, j)),
            pl.BlockSpec((1, BN), lambda i, j: (0, j)),
        ],
        out_specs=pl.BlockSpec((BM, BN), lambda i, j: (i, j)),
        out_shape=jax.ShapeDtypeStruct((M, N), jnp.float32),
        compiler_params=pltpu.CompilerParams(
            dimension_semantics=("parallel", "parallel"),
        ),
    )(inputs, w16, b2d)
    return out
